# baseline (speedup 1.0000x reference)
import jax
import jax.numpy as jnp
from jax.experimental import pallas as pl

_TOP_K = 8


def _router_body(xa_ref, xb_ref, w_ref, pa_ref, ia_ref, pb_ref, ib_ref):
    w = w_ref[...]
    sa = jnp.dot(xa_ref[...], w, preferred_element_type=jnp.float32)
    sb = jnp.dot(xb_ref[...], w, preferred_element_type=jnp.float32)
    pa_ref[...] = sa[:, :_TOP_K]
    pb_ref[...] = sb[:, :_TOP_K]
    ia_ref[...] = jnp.zeros_like(ia_ref)
    ib_ref[...] = jnp.zeros_like(ib_ref)


def kernel(x, W_gate):
    b, s, d = x.shape
    e = W_gate.shape[0]
    t = b * s
    h = t // 2
    xf = x.reshape(t, d)
    xa = xf[:h]
    xb = xf[h:]
    wt = W_gate.T
    bt = 512
    grid = (h // bt,)
    pa, ia, pb, ib = pl.pallas_call(
        _router_body,
        grid=grid,
        in_specs=[
            pl.BlockSpec((bt, d), lambda i: (i, 0)),
            pl.BlockSpec((bt, d), lambda i: (i, 0)),
            pl.BlockSpec((d, e), lambda i: (0, 0)),
        ],
        out_specs=[
            pl.BlockSpec((bt, _TOP_K), lambda i: (i, 0)),
            pl.BlockSpec((bt, _TOP_K), lambda i: (i, 0)),
            pl.BlockSpec((bt, _TOP_K), lambda i: (i, 0)),
            pl.BlockSpec((bt, _TOP_K), lambda i: (i, 0)),
        ],
        out_shape=[
            jax.ShapeDtypeStruct((h, _TOP_K), jnp.float32),
            jax.ShapeDtypeStruct((h, _TOP_K), jnp.int32),
            jax.ShapeDtypeStruct((h, _TOP_K), jnp.float32),
            jax.ShapeDtypeStruct((h, _TOP_K), jnp.int32),
        ],
    )(xa, xb, wt)
    probs = jnp.concatenate([pa, pb], axis=0).reshape(b, s, _TOP_K)
    idx = jnp.concatenate([ia, ib], axis=0).reshape(b, s, _TOP_K)
    return probs, idx


# speedup vs baseline: 2.4643x; 2.4643x over previous
import jax
import jax.numpy as jnp
from jax.experimental import pallas as pl

_TOP_K = 8


def _topk_softmax_chunk(s, iota, ef):
    vals = []
    idxs = []
    for k in range(_TOP_K):
        m = jnp.max(s, axis=1, keepdims=True)
        eq = s == m
        hit = jnp.where(eq, iota, ef)
        idx = jnp.min(hit, axis=1, keepdims=True)
        vals.append(m)
        idxs.append(idx)
        if k + 1 < _TOP_K:
            s = jnp.where(eq, -jnp.inf, s)
    v = jnp.concatenate(vals, axis=1)
    ix = jnp.concatenate(idxs, axis=1)
    ex = jnp.exp(v - v[:, 0:1])
    return ex / jnp.sum(ex, axis=1, keepdims=True), ix.astype(jnp.int32)


def _router_body(x_ref, w_ref, probs_ref, idx_ref):
    bt = x_ref.shape[0]
    e = w_ref.shape[1]
    h = bt // 2
    w = w_ref[...]
    rc = 64
    iota = jax.lax.broadcasted_iota(jnp.int32, (rc, e), 1).astype(jnp.float32)
    ef = float(e)
    s1 = jnp.dot(x_ref[0:h, :], w, preferred_element_type=jnp.float32)
    nchunks = h // rc
    s2_pieces = []
    out1 = []
    for c in range(nchunks):
        lo = h + c * rc
        s2_pieces.append(
            jnp.dot(x_ref[lo:lo + rc, :], w, preferred_element_type=jnp.float32))
        out1.append(_topk_softmax_chunk(s1[c * rc:(c + 1) * rc, :], iota, ef))
    for c in range(nchunks):
        p, ix = out1[c]
        probs_ref[c * rc:(c + 1) * rc, :] = p
        idx_ref[c * rc:(c + 1) * rc, :] = ix
        p2, ix2 = _topk_softmax_chunk(s2_pieces[c], iota, ef)
        lo = h + c * rc
        probs_ref[lo:lo + rc, :] = p2
        idx_ref[lo:lo + rc, :] = ix2


def kernel(x, W_gate):
    b, s, d = x.shape
    e = W_gate.shape[0]
    t = b * s
    xf = x.reshape(t, d)
    wt = W_gate.T
    bt = min(1024, t)
    grid = (t // bt,)
    probs, idx = pl.pallas_call(
        _router_body,
        grid=grid,
        in_specs=[
            pl.BlockSpec((bt, d), lambda i: (i, 0)),
            pl.BlockSpec((d, e), lambda i: (0, 0)),
        ],
        out_specs=[
            pl.BlockSpec((bt, _TOP_K), lambda i: (i, 0)),
            pl.BlockSpec((bt, _TOP_K), lambda i: (i, 0)),
        ],
        out_shape=[
            jax.ShapeDtypeStruct((t, _TOP_K), jnp.float32),
            jax.ShapeDtypeStruct((t, _TOP_K), jnp.int32),
        ],
    )(xf, wt)
    return probs.reshape(b, s, _TOP_K), idx.reshape(b, s, _TOP_K)
